# baseline (speedup 1.0000x reference)
import functools

import jax
import jax.numpy as jnp
from jax import lax
from jax.experimental import pallas as pl
from jax.experimental.pallas import tpu as pltpu
from jax.experimental.pallas import tpu_sc as plsc

DIM = 32
NC = 2
NS = 16
NW = NC * NS


def _make_gather(b, l):
    bpw = b // NW
    rows_pw = (bpw * l) // 128
    nbk = bpw // 128

    mesh = plsc.VectorSubcoreMesh(core_axis_name="c", subcore_axis_name="s")

    @functools.partial(
        pl.kernel,
        mesh=mesh,
        out_type=jax.ShapeDtypeStruct((l * DIM * b,), jnp.float32),
        scratch_types=[
            pltpu.VMEM((rows_pw, 128), jnp.int32),
            pltpu.VMEM((l, bpw), jnp.int32),
            pltpu.VMEM((2, bpw, DIM), jnp.float32),
            pltpu.VMEM((2, 2, 2 * nbk * 1024), jnp.float32),
            pltpu.VMEM((528,), jnp.float32),
            pltpu.SemaphoreType.DMA((2,)),
            pltpu.SemaphoreType.DMA((2,)),
        ],
        compiler_params=pltpu.CompilerParams(
            use_tc_tiling_on_sc=False, needs_layout_passes=False
        ),
    )
    def gather_kernel(idx_hbm, table_hbm, out_hbm, idxv, lidx, rows, tbuf,
                      pad, gsem, osem):
        wid = lax.axis_index("s") * NC + lax.axis_index("c")
        iota = lax.iota(jnp.int32, 16)

        pltpu.sync_copy(idx_hbm.at[pl.ds(wid * rows_pw, rows_pw)], idxv)

        v_l = iota * l

        def repack(li, _):
            def inner(c, _):
                p = v_l + (c * 16 * l + li)
                r = lax.shift_right_logical(p, 7)
                cc = lax.bitwise_and(p, 127)
                lidx[li, pl.ds(c * 16, 16)] = plsc.load_gather(idxv, [r, cc])
                return 0

            lax.fori_loop(0, bpw // 16, inner, 0)
            return 0

        lax.fori_loop(0, l, repack, 0)

        def fire_gathers(li, p):
            pltpu.async_copy(
                table_hbm.at[lidx.at[li]],
                rows.at[p],
                gsem.at[p],
            )

        def drain_gathers(p):
            pltpu.make_async_copy(
                table_hbm.at[lidx.at[0]],
                rows.at[p],
                gsem.at[p],
            ).wait()

        gsz = nbk * 1024

        def fire_writes(li, p):
            for g in range(DIM // 8):
                off = li * b * DIM + g * b * 8 + wid * gsz
                pltpu.async_copy(
                    tbuf.at[p, g // 2, pl.ds((g % 2) * gsz, gsz)],
                    out_hbm.at[pl.ds(off, gsz)],
                    osem.at[p],
                )

        def drain_writes(li, p):
            for g in range(DIM // 8):
                off = li * b * DIM + g * b * 8 + wid * gsz
                pltpu.make_async_copy(
                    tbuf.at[p, g // 2, pl.ds((g % 2) * gsz, gsz)],
                    out_hbm.at[pl.ds(off, gsz)],
                    osem.at[p],
                ).wait()

        col_pos = [iota * 33 + (d if d < 16 else 17 + (d - 16)) for d in range(DIM)]

        def transpose(p):
            def blk_body(blk, _):
                bq0 = blk * 16
                base = lax.shift_right_logical(bq0, 7) * 1024 + lax.bitwise_and(bq0, 127)
                for i in range(16):
                    for h in range(2):
                        pad[pl.ds(i * 33 + h * 17, 16)] = rows[p, bq0 + i, pl.ds(h * 16, 16)]
                for d in range(DIM):
                    v = plsc.load_gather(pad, [col_pos[d]])
                    off = (d // 8 % 2) * 4096 + (d % 8) * 128 + base
                    tbuf[p, d // 16, pl.ds(off, 16)] = v
                return 0

            lax.fori_loop(0, bpw // 16, blk_body, 0)

        fire_gathers(0, 0)

        def body(ll, _):
            for sub in range(2):
                li = ll * 2 + sub
                p = sub

                @pl.when(li + 1 < l)
                def _():
                    fire_gathers(li + 1, 1 - p)

                drain_gathers(p)

                @pl.when(ll >= 1)
                def _():
                    drain_writes(li - 2, p)

                transpose(p)
                fire_writes(li, p)
            return 0

        lax.fori_loop(0, l // 2, body, 0)

        drain_writes(l - 2, 0)
        drain_writes(l - 1, 1)

    return gather_kernel


def kernel(input_words, in_table):
    b, l = input_words.shape
    n = b * l
    idx2d = input_words.reshape(n // 128, 128).astype(jnp.int32)
    y = _make_gather(b, l)(idx2d, in_table)
    y5 = y.reshape(l, DIM // 8, b // 128, 8, 128)
    t = y5.transpose(2, 4, 0, 1, 3)
    return t.reshape(b, l, DIM)


# speedup vs baseline: 1.0336x; 1.0336x over previous
import functools

import jax
import jax.numpy as jnp
from jax import lax
from jax.experimental import pallas as pl
from jax.experimental.pallas import tpu as pltpu
from jax.experimental.pallas import tpu_sc as plsc

DIM = 32
NC = 2
NS = 16
NW = NC * NS


def _make_gather(b, l):
    bpw = b // NW
    rows_pw = (bpw * l) // 128
    nbk = bpw // 128

    mesh = plsc.VectorSubcoreMesh(core_axis_name="c", subcore_axis_name="s")

    @functools.partial(
        pl.kernel,
        mesh=mesh,
        out_type=jax.ShapeDtypeStruct((l * DIM * b,), jnp.float32),
        scratch_types=[
            pltpu.VMEM((rows_pw, 128), jnp.int32),
            pltpu.VMEM((l, bpw), jnp.int32),
            pltpu.VMEM((2, bpw, DIM), jnp.float32),
            pltpu.VMEM((2, 2, 2 * nbk * 1024), jnp.float32),
            pltpu.SemaphoreType.DMA((2,)),
            pltpu.SemaphoreType.DMA((2,)),
        ],
        compiler_params=pltpu.CompilerParams(
            use_tc_tiling_on_sc=False, needs_layout_passes=False
        ),
    )
    def gather_kernel(idx_hbm, table_hbm, out_hbm, idxv, lidx, rows, tbuf,
                      gsem, osem):
        wid = lax.axis_index("s") * NC + lax.axis_index("c")
        iota = lax.iota(jnp.int32, 16)

        pltpu.sync_copy(idx_hbm.at[pl.ds(wid * rows_pw, rows_pw)], idxv)

        v_l = iota * l

        def repack(li, _):
            def inner(c, _):
                p = v_l + (c * 16 * l + li)
                r = lax.shift_right_logical(p, 7)
                cc = lax.bitwise_and(p, 127)
                lidx[li, pl.ds(c * 16, 16)] = plsc.load_gather(idxv, [r, cc])
                return 0

            lax.fori_loop(0, bpw // 16, inner, 0)
            return 0

        lax.fori_loop(0, l, repack, 0)

        def fire_gathers(li, p):
            pltpu.async_copy(
                table_hbm.at[lidx.at[li]],
                rows.at[p],
                gsem.at[p],
            )

        def drain_gathers(p):
            pltpu.make_async_copy(
                table_hbm.at[lidx.at[0]],
                rows.at[p],
                gsem.at[p],
            ).wait()

        gsz = nbk * 1024

        def fire_writes(li, p):
            for g in range(DIM // 8):
                off = li * b * DIM + g * b * 8 + wid * gsz
                pltpu.async_copy(
                    tbuf.at[p, g // 2, pl.ds((g % 2) * gsz, gsz)],
                    out_hbm.at[pl.ds(off, gsz)],
                    osem.at[p],
                )

        def drain_writes(li, p):
            for g in range(DIM // 8):
                off = li * b * DIM + g * b * 8 + wid * gsz
                pltpu.make_async_copy(
                    tbuf.at[p, g // 2, pl.ds((g % 2) * gsz, gsz)],
                    out_hbm.at[pl.ds(off, gsz)],
                    osem.at[p],
                ).wait()

        pb = lax.shift_right_logical(iota, 3) * 4096 + lax.bitwise_and(iota, 7) * 128

        def transpose(p):
            @plsc.parallel_loop(0, bpw, unroll=8)
            def _(bq):
                base = lax.shift_right_logical(bq, 7) * 1024 + lax.bitwise_and(bq, 127)
                pos = pb + base
                for h in range(2):
                    v = rows[p, bq, pl.ds(h * 16, 16)]
                    plsc.store_scatter(tbuf.at[p, h], [pos], v)

        fire_gathers(0, 0)

        def body(ll, _):
            for sub in range(2):
                li = ll * 2 + sub
                p = sub

                @pl.when(li + 1 < l)
                def _():
                    fire_gathers(li + 1, 1 - p)

                drain_gathers(p)

                @pl.when(ll >= 1)
                def _():
                    drain_writes(li - 2, p)

                transpose(p)
                fire_writes(li, p)
            return 0

        lax.fori_loop(0, l // 2, body, 0)

        drain_writes(l - 2, 0)
        drain_writes(l - 1, 1)

    return gather_kernel


def kernel(input_words, in_table):
    b, l = input_words.shape
    n = b * l
    idx2d = input_words.reshape(n // 128, 128).astype(jnp.int32)
    y = _make_gather(b, l)(idx2d, in_table)
    y5 = y.reshape(l, DIM // 8, b // 128, 8, 128)
    t = y5.transpose(2, 4, 0, 1, 3)
    return t.reshape(b, l, DIM)


# speedup vs baseline: 1.4961x; 1.4474x over previous
import functools

import jax
import jax.numpy as jnp
from jax import lax
from jax.experimental import pallas as pl
from jax.experimental.pallas import tpu as pltpu
from jax.experimental.pallas import tpu_sc as plsc

DIM = 32
NC = 2
NS = 16
NW = NC * NS


def _make_gather(b, l):
    bpw = b // NW
    rows_pw = (bpw * l) // 128
    nbk = bpw // 128

    mesh = plsc.VectorSubcoreMesh(core_axis_name="c", subcore_axis_name="s")

    @functools.partial(
        pl.kernel,
        mesh=mesh,
        out_type=jax.ShapeDtypeStruct((l * DIM * b // 128, 128), jnp.float32),
        scratch_types=[
            pltpu.VMEM((rows_pw, 128), jnp.int32),
            pltpu.VMEM((l, bpw), jnp.int32),
            pltpu.VMEM((2, bpw, DIM), jnp.float32),
            pltpu.VMEM((2, 2, 2 * nbk * 8, 137), jnp.float32),
            pltpu.SemaphoreType.DMA((2,)),
            pltpu.SemaphoreType.DMA((2,)),
        ],
        compiler_params=pltpu.CompilerParams(
            use_tc_tiling_on_sc=False, needs_layout_passes=False
        ),
    )
    def gather_kernel(idx_hbm, table_hbm, out_hbm, idxv, lidx, rows, tbuf,
                      gsem, osem):
        wid = lax.axis_index("s") * NC + lax.axis_index("c")
        iota = lax.iota(jnp.int32, 16)

        pltpu.sync_copy(idx_hbm.at[pl.ds(wid * rows_pw, rows_pw)], idxv)

        v_l = iota * l

        def repack(li, _):
            def inner(c, _):
                p = v_l + (c * 16 * l + li)
                r = lax.shift_right_logical(p, 7)
                cc = lax.bitwise_and(p, 127)
                lidx[li, pl.ds(c * 16, 16)] = plsc.load_gather(idxv, [r, cc])
                return 0

            lax.fori_loop(0, bpw // 16, inner, 0)
            return 0

        lax.fori_loop(0, l, repack, 0)

        def fire_gathers(li, p):
            pltpu.async_copy(
                table_hbm.at[lidx.at[li]],
                rows.at[p],
                gsem.at[p],
            )

        def drain_gathers(p):
            pltpu.make_async_copy(
                table_hbm.at[lidx.at[0]],
                rows.at[p],
                gsem.at[p],
            ).wait()

        grows = nbk * 8

        def fire_writes(li, p):
            for g in range(DIM // 8):
                row0 = (li * b * DIM + g * b * 8 + wid * grows * 128) // 128
                pltpu.async_copy(
                    tbuf.at[p, g // 2, pl.ds((g % 2) * grows, grows), pl.ds(0, 128)],
                    out_hbm.at[pl.ds(row0, grows)],
                    osem.at[p],
                )

        def drain_writes(li, p):
            for g in range(DIM // 8):
                row0 = (li * b * DIM + g * b * 8 + wid * grows * 128) // 128
                pltpu.make_async_copy(
                    tbuf.at[p, g // 2, pl.ds((g % 2) * grows, grows), pl.ds(0, 128)],
                    out_hbm.at[pl.ds(row0, grows)],
                    osem.at[p],
                ).wait()

        pb = lax.shift_right_logical(iota, 3) * 32 + lax.bitwise_and(iota, 7)

        def transpose(p):
            @plsc.parallel_loop(0, bpw, unroll=8)
            def _(bq):
                rowv = pb + lax.shift_right_logical(bq, 7) * 8
                colv = iota * 0 + lax.bitwise_and(bq, 127)
                for h in range(2):
                    v = rows[p, bq, pl.ds(h * 16, 16)]
                    plsc.store_scatter(tbuf.at[p, h], [rowv, colv], v)

        fire_gathers(0, 0)

        def body(ll, _):
            for sub in range(2):
                li = ll * 2 + sub
                p = sub

                @pl.when(li + 1 < l)
                def _():
                    fire_gathers(li + 1, 1 - p)

                drain_gathers(p)

                @pl.when(ll >= 1)
                def _():
                    drain_writes(li - 2, p)

                transpose(p)
                fire_writes(li, p)
            return 0

        lax.fori_loop(0, l // 2, body, 0)

        drain_writes(l - 2, 0)
        drain_writes(l - 1, 1)

    return gather_kernel


def kernel(input_words, in_table):
    b, l = input_words.shape
    n = b * l
    idx2d = input_words.reshape(n // 128, 128).astype(jnp.int32)
    y = _make_gather(b, l)(idx2d, in_table)
    y5 = y.reshape(l, DIM // 8, b // 128, 8, 128)
    t = y5.transpose(2, 4, 0, 1, 3)
    return t.reshape(b, l, DIM)
